# Initial kernel scaffold; baseline (speedup 1.0000x reference)
import jax
import jax.numpy as jnp
from jax.experimental import pallas as pl


def kernel(points, features, W0, b0, W1, b1, W2, b2, Wsc, bsc):
    raise NotImplementedError("write your pallas kernel here")


# speedup vs baseline: 11.3934x; 11.3934x over previous
import functools

import jax
import jax.numpy as jnp
from jax import lax
from jax.experimental import pallas as pl
from jax.experimental.pallas import tpu as pltpu
from jax.experimental.pallas import tpu_sc as plsc

_K = 16


def _topk_prep_body(n_total, rows, d, pts_ref, ptsT_ref, f_ref, w0_ref,
                    idx_ref, g_ref, h_ref):
    b = pl.program_id(0)
    ptsT = ptsT_ref[0]
    r_row = jnp.sum(ptsT * ptsT, axis=0, keepdims=True)
    p = pts_ref[0]
    r_col = jnp.sum(p * p, axis=1, keepdims=True)
    m = lax.dot_general(p.astype(jnp.bfloat16), ptsT.astype(jnp.bfloat16),
                        (((1,), (0,)), ((), ())),
                        preferred_element_type=jnp.float32)
    S = (r_col - 2.0 * m) + r_row
    col = lax.broadcasted_iota(jnp.int32, S.shape, 1)
    inf = jnp.float32(jnp.inf)
    cols = []
    for _ in range(_K + 1):
        mn = jnp.min(S, axis=1, keepdims=True)
        j = jnp.min(jnp.where(S == mn, col, n_total), axis=1)
        cols.append(j[:, None])
        S = jnp.where(col == j[:, None], inf, S)
    idx_ref[0] = jnp.concatenate(cols[1:], axis=1) + b * n_total
    wg = w0_ref[0:d, :] + w0_ref[d:2 * d, :]
    wh = w0_ref[d:2 * d, :]
    f = f_ref[0]
    g_ref[0] = jnp.dot(f, wg, preferred_element_type=jnp.float32)
    h_ref[0] = jnp.dot(f, wh, preferred_element_type=jnp.float32)


def _mlp_body(d, knn_ref, h_ref, b0_ref, w1_ref, b1_ref, w2_ref, b2_ref,
              wsc_ref, bsc_ref, out_ref):
    rn = knn_ref.shape[0]
    x = knn_ref[...]
    h = h_ref[...]
    x0 = jnp.maximum(x - h[:, None, :] + b0_ref[...][None], 0.0)
    x0 = x0.reshape(rn * _K, d)
    x1 = jnp.maximum(
        jnp.dot(x0, w1_ref[...], preferred_element_type=jnp.float32)
        + b1_ref[...], 0.0)
    x2 = jnp.maximum(
        jnp.dot(x1, w2_ref[...], preferred_element_type=jnp.float32)
        + b2_ref[...], 0.0)
    feats = jnp.sum(x2.reshape(rn, _K, d), axis=1) * (1.0 / _K)
    sc = jnp.dot(feats, wsc_ref[...], preferred_element_type=jnp.float32) + bsc_ref[...]
    out_ref[...] = jnp.maximum(sc + feats, 0.0)


def _make_sc_gather(total, d, chunk):
    info = plsc.get_sparse_core_info()
    nc, ns = info.num_cores, info.num_subcores
    nw = nc * ns
    per_w = total // nw
    iters = per_w // chunk
    mesh = plsc.VectorSubcoreMesh(core_axis_name="c", subcore_axis_name="s")

    @functools.partial(
        pl.kernel, mesh=mesh,
        out_type=jax.ShapeDtypeStruct((total, d), jnp.float32),
        scratch_types=[
            pltpu.VMEM((chunk,), jnp.int32),
            pltpu.VMEM((chunk, d), jnp.float32),
            pltpu.SemaphoreType.DMA,
        ],
    )
    def gather(idx_hbm, table_hbm, out_hbm, idx_v, rows_v, sem):
        wid = lax.axis_index("s") * nc + lax.axis_index("c")
        base = wid * per_w

        def body(t, c):
            off = base + t * chunk
            pltpu.sync_copy(idx_hbm.at[pl.ds(off, chunk)], idx_v)
            pltpu.async_copy(table_hbm.at[idx_v], rows_v, sem).wait()
            pltpu.sync_copy(rows_v, out_hbm.at[pl.ds(off, chunk)])
            return c

        lax.fori_loop(0, iters, body, 0)

    return gather


def kernel(points, features, W0, b0, W1, b1, W2, b2, Wsc, bsc):
    B, N, _ = points.shape
    d = features.shape[-1]
    R = 256
    pts = jnp.concatenate(
        [points, jnp.zeros((B, N, 5), points.dtype)], axis=-1)
    ptsT = jnp.transpose(pts, (0, 2, 1))

    idx, G, H = pl.pallas_call(
        functools.partial(_topk_prep_body, N, R, d),
        grid=(B, N // R),
        in_specs=[
            pl.BlockSpec((1, R, 8), lambda b, i: (b, i, 0)),
            pl.BlockSpec((1, 8, N), lambda b, i: (b, 0, 0)),
            pl.BlockSpec((1, R, d), lambda b, i: (b, i, 0)),
            pl.BlockSpec((2 * d, d), lambda b, i: (0, 0)),
        ],
        out_specs=[
            pl.BlockSpec((1, R, _K), lambda b, i: (b, i, 0)),
            pl.BlockSpec((1, R, d), lambda b, i: (b, i, 0)),
            pl.BlockSpec((1, R, d), lambda b, i: (b, i, 0)),
        ],
        out_shape=[
            jax.ShapeDtypeStruct((B, N, _K), jnp.int32),
            jax.ShapeDtypeStruct((B, N, d), jnp.float32),
            jax.ShapeDtypeStruct((B, N, d), jnp.float32),
        ],
    )(pts, ptsT, features, W0)

    total = B * N * _K
    knn = _make_sc_gather(total, d, 128)(
        idx.reshape(total), G.reshape(B * N, d))

    RN = 256
    out = pl.pallas_call(
        functools.partial(_mlp_body, d),
        grid=(B * N // RN,),
        in_specs=[
            pl.BlockSpec((RN, _K, d), lambda i: (i, 0, 0)),
            pl.BlockSpec((RN, d), lambda i: (i, 0)),
            pl.BlockSpec((1, d), lambda i: (0, 0)),
            pl.BlockSpec((d, d), lambda i: (0, 0)),
            pl.BlockSpec((1, d), lambda i: (0, 0)),
            pl.BlockSpec((d, d), lambda i: (0, 0)),
            pl.BlockSpec((1, d), lambda i: (0, 0)),
            pl.BlockSpec((d, d), lambda i: (0, 0)),
            pl.BlockSpec((1, d), lambda i: (0, 0)),
        ],
        out_specs=pl.BlockSpec((RN, d), lambda i: (i, 0)),
        out_shape=jax.ShapeDtypeStruct((B * N, d), jnp.float32),
    )(knn.reshape(B * N, _K, d), H.reshape(B * N, d),
      b0.reshape(1, d), W1, b1.reshape(1, d), W2, b2.reshape(1, d),
      Wsc, bsc.reshape(1, d))
    return out.reshape(B, N, d)


# speedup vs baseline: 16.4742x; 1.4459x over previous
import functools

import jax
import jax.numpy as jnp
from jax import lax
from jax.experimental import pallas as pl
from jax.experimental.pallas import tpu as pltpu
from jax.experimental.pallas import tpu_sc as plsc

_K = 16


def _topk_prep_body(n_total, rows, d, pts_ref, ptsT_ref, f_ref, w0_ref,
                    idx_ref, g_ref, h_ref):
    b = pl.program_id(0)
    ptsT = ptsT_ref[0]
    r_row = jnp.sum(ptsT * ptsT, axis=0, keepdims=True)
    p = pts_ref[0]
    r_col = jnp.sum(p * p, axis=1, keepdims=True)
    m = lax.dot_general(p.astype(jnp.bfloat16), ptsT.astype(jnp.bfloat16),
                        (((1,), (0,)), ((), ())),
                        preferred_element_type=jnp.float32)
    S = (r_col - 2.0 * m) + r_row
    colf = lax.broadcasted_iota(jnp.int32, S.shape, 1).astype(jnp.float32)
    inf = jnp.float32(jnp.inf)
    bigf = jnp.float32(n_total)
    cols = []
    for t in range(_K + 1):
        mn = jnp.min(S, axis=1, keepdims=True)
        jf = jnp.min(jnp.where(S == mn, colf, bigf), axis=1, keepdims=True)
        cols.append(jf)
        if t < _K:
            S = jnp.where(colf == jf, inf, S)
    idx_ref[0] = (jnp.concatenate(cols[1:], axis=1)
                  + jnp.float32(b * n_total)).astype(jnp.int32)
    wg = w0_ref[0:d, :] + w0_ref[d:2 * d, :]
    wh = w0_ref[d:2 * d, :]
    f = f_ref[0]
    g_ref[0] = jnp.dot(f, wg, preferred_element_type=jnp.float32)
    h_ref[0] = jnp.dot(f, wh, preferred_element_type=jnp.float32)


def _mlp_body(d, knn_ref, h_ref, b0_ref, w1_ref, b1_ref, w2_ref, b2_ref,
              wsc_ref, bsc_ref, out_ref):
    rn = knn_ref.shape[0]
    x = knn_ref[...]
    h = h_ref[...]
    x0 = jnp.maximum(x - h[:, None, :] + b0_ref[...][None], 0.0)
    x0 = x0.reshape(rn * _K, d)
    x1 = jnp.maximum(
        jnp.dot(x0, w1_ref[...], preferred_element_type=jnp.float32)
        + b1_ref[...], 0.0)
    x2 = jnp.maximum(
        jnp.dot(x1, w2_ref[...], preferred_element_type=jnp.float32)
        + b2_ref[...], 0.0)
    feats = jnp.sum(x2.reshape(rn, _K, d), axis=1) * (1.0 / _K)
    sc = jnp.dot(feats, wsc_ref[...], preferred_element_type=jnp.float32) + bsc_ref[...]
    out_ref[...] = jnp.maximum(sc + feats, 0.0)


def _make_sc_gather(total, d, chunk):
    info = plsc.get_sparse_core_info()
    nc, ns = info.num_cores, info.num_subcores
    nw = nc * ns
    per_w = total // nw
    iters = per_w // chunk
    mesh = plsc.VectorSubcoreMesh(core_axis_name="c", subcore_axis_name="s")

    @functools.partial(
        pl.kernel, mesh=mesh,
        out_type=jax.ShapeDtypeStruct((total, d), jnp.float32),
        scratch_types=[
            pltpu.VMEM((chunk,), jnp.int32),
            pltpu.VMEM((chunk, d), jnp.float32),
            pltpu.SemaphoreType.DMA,
        ],
    )
    def gather(idx_hbm, table_hbm, out_hbm, idx_v, rows_v, sem):
        wid = lax.axis_index("s") * nc + lax.axis_index("c")
        base = wid * per_w

        def body(t, c):
            off = base + t * chunk
            pltpu.sync_copy(idx_hbm.at[pl.ds(off, chunk)], idx_v)
            pltpu.async_copy(table_hbm.at[idx_v], rows_v, sem).wait()
            pltpu.sync_copy(rows_v, out_hbm.at[pl.ds(off, chunk)])
            return c

        lax.fori_loop(0, iters, body, 0)

    return gather


def kernel(points, features, W0, b0, W1, b1, W2, b2, Wsc, bsc):
    B, N, _ = points.shape
    d = features.shape[-1]
    R = 256
    pts = jnp.concatenate(
        [points, jnp.zeros((B, N, 5), points.dtype)], axis=-1)
    ptsT = jnp.transpose(pts, (0, 2, 1))

    topk_prep = pl.pallas_call(
        functools.partial(_topk_prep_body, N, R, d),
        grid=(1, N // R),
        in_specs=[
            pl.BlockSpec((1, R, 8), lambda b, i: (b, i, 0)),
            pl.BlockSpec((1, 8, N), lambda b, i: (b, 0, 0)),
            pl.BlockSpec((1, R, d), lambda b, i: (b, i, 0)),
            pl.BlockSpec((2 * d, d), lambda b, i: (0, 0)),
        ],
        out_specs=[
            pl.BlockSpec((1, R, _K), lambda b, i: (b, i, 0)),
            pl.BlockSpec((1, R, d), lambda b, i: (b, i, 0)),
            pl.BlockSpec((1, R, d), lambda b, i: (b, i, 0)),
        ],
        out_shape=[
            jax.ShapeDtypeStruct((1, N, _K), jnp.int32),
            jax.ShapeDtypeStruct((1, N, d), jnp.float32),
            jax.ShapeDtypeStruct((1, N, d), jnp.float32),
        ],
    )

    total = N * _K
    gather = _make_sc_gather(total, d, 128)

    RN = 256
    mlp = pl.pallas_call(
        functools.partial(_mlp_body, d),
        grid=(N // RN,),
        in_specs=[
            pl.BlockSpec((RN, _K, d), lambda i: (i, 0, 0)),
            pl.BlockSpec((RN, d), lambda i: (i, 0)),
            pl.BlockSpec((1, d), lambda i: (0, 0)),
            pl.BlockSpec((d, d), lambda i: (0, 0)),
            pl.BlockSpec((1, d), lambda i: (0, 0)),
            pl.BlockSpec((d, d), lambda i: (0, 0)),
            pl.BlockSpec((1, d), lambda i: (0, 0)),
            pl.BlockSpec((d, d), lambda i: (0, 0)),
            pl.BlockSpec((1, d), lambda i: (0, 0)),
        ],
        out_specs=pl.BlockSpec((RN, d), lambda i: (i, 0)),
        out_shape=jax.ShapeDtypeStruct((N, d), jnp.float32),
    )

    b0r, b1r, b2r, bscr = (b0.reshape(1, d), b1.reshape(1, d),
                           b2.reshape(1, d), bsc.reshape(1, d))
    outs = []
    for b in range(B):
        idx_b, G_b, H_b = topk_prep(pts[b:b + 1], ptsT[b:b + 1],
                                    features[b:b + 1], W0)
        knn_b = gather(idx_b.reshape(total), G_b.reshape(N, d))
        outs.append(mlp(knn_b.reshape(N, _K, d), H_b.reshape(N, d),
                        b0r, W1, b1r, W2, b2r, Wsc, bscr))
    return jnp.stack(outs)
